# baseline (speedup 1.0000x reference)
import jax
import jax.numpy as jnp
import numpy as np
from jax import lax
from jax.experimental import pallas as pl
from jax.experimental.pallas import tpu as pltpu
from jax.experimental.pallas import tpu_sc as plsc

_BATCH = 16384
_NF = 26
_D = 16
_FIELD_DIM = 38462
_FLAT = _BATCH * _NF
_NC = 2
_NS = 16
_NW = _NC * _NS
_EPW = _FLAT // _NW
_CH = 128
_NCH = _EPW // _CH
_NBUF = 8
_LANES = 16

_OFF_PATTERN = np.reshape(
    (np.arange(_EPW, dtype=np.int64) % _NF) * _FIELD_DIM, (_NCH, _CH)
).astype(np.int32)


def _body(idx_hbm, off_hbm, table_hbm, out_hbm, idx_v, off_v, rows_v, *sems):
  gsem = sems[:_NBUF]
  wsem = sems[_NBUF:]
  wid = lax.axis_index("s") * _NC + lax.axis_index("c")
  base_chunk = wid * _NCH
  base_row = wid * _EPW

  pltpu.sync_copy(idx_hbm.at[pl.ds(base_chunk, _NCH)], idx_v)
  pltpu.sync_copy(off_hbm, off_v)

  def add_offsets(c):
    for k in range(_CH // _LANES):
      sl = pl.ds(k * _LANES, _LANES)
      idx_v[c, sl] = idx_v[c, sl] + off_v[c, sl]

  def fire_gather(c, b):
    pltpu.make_async_copy(
        table_hbm.at[idx_v.at[c]], rows_v.at[b], gsem[b]
    ).start()

  def wait_gather(c, b):
    pltpu.make_async_copy(
        table_hbm.at[idx_v.at[c]], rows_v.at[b], gsem[b]
    ).wait()

  def fire_writeback(c, b):
    pltpu.make_async_copy(
        rows_v.at[b], out_hbm.at[pl.ds(base_row + c * _CH, _CH)], wsem[b]
    ).start()

  def wait_writeback(c, b):
    pltpu.make_async_copy(
        rows_v.at[b], out_hbm.at[pl.ds(base_row + c * _CH, _CH)], wsem[b]
    ).wait()

  for b in range(_NBUF):
    add_offsets(b)
    fire_gather(b, b)

  def round_body(r, carry):
    for b in range(_NBUF):
      c = r * _NBUF + b
      wait_gather(c, b)
      fire_writeback(c, b)
      wait_writeback(c, b)
      n = c + _NBUF

      @pl.when(n < _NCH)
      def _():
        add_offsets(n)
        fire_gather(n, b)

    return carry

  lax.fori_loop(0, _NCH // _NBUF, round_body, 0)


@jax.jit
def kernel(x, table):
  idx = jnp.reshape(x.astype(jnp.int32), (_NW * _NCH, _CH))
  off = jnp.asarray(_OFF_PATTERN)
  mesh = plsc.VectorSubcoreMesh(core_axis_name="c", subcore_axis_name="s")
  scratch = [
      pltpu.VMEM((_NCH, _CH), jnp.int32),
      pltpu.VMEM((_NCH, _CH), jnp.int32),
      pltpu.VMEM((_NBUF, _CH, _D), jnp.float32),
  ] + [pltpu.SemaphoreType.DMA] * (2 * _NBUF)
  out = pl.kernel(
      _body,
      out_type=jax.ShapeDtypeStruct((_FLAT, _D), jnp.float32),
      mesh=mesh,
      scratch_types=scratch,
      compiler_params=pltpu.CompilerParams(use_tc_tiling_on_sc=False),
  )(idx, off, table)
  return jnp.reshape(out, (_BATCH, _NF, _D))
